# Initial kernel scaffold; baseline (speedup 1.0000x reference)
import jax
import jax.numpy as jnp
from jax.experimental import pallas as pl


def kernel(context, A0, A1, A2, C_last):
    raise NotImplementedError("write your pallas kernel here")


# speedup vs baseline: 26.1137x; 26.1137x over previous
import functools

import jax
import jax.numpy as jnp
from jax import lax
from jax.experimental import pallas as pl
from jax.experimental.pallas import tpu as pltpu
from jax.experimental.pallas import tpu_sc as plsc

B, M, S, E = 1024, 50, 20, 32

NC = 2
NS = 16
NW = NC * NS

ROWS = B * M
RPW = ROWS // NW
CHUNK = 32
NCH = RPW // CHUNK
IPC = CHUNK * S
NSTR = IPC // 128

def _seg_gather_body(ctx_hbm, t0_hbm, t1_hbm, t2_hbm, o0_hbm, o1_hbm, o2_hbm,
                     idx_v, rows_v, out_v, gsem0, gsem1):
  wid = lax.axis_index("s") * NC + lax.axis_index("c")
  idx_base = wid * (NCH * IPC)

  def load_idx(c, buf):
    pltpu.sync_copy(ctx_hbm.at[pl.ds(idx_base + c * IPC, IPC)],
                    idx_v.at[buf])

  def fire(tbl, buf, sem):
    for k in range(NSTR):
      pltpu.async_copy(tbl.at[idx_v.at[buf, pl.ds(k * 128, 128)]],
                       rows_v.at[buf, pl.ds(k * 128, 128), :],
                       sem)

  def drain(tbl, buf, sem):
    for k in range(NSTR):
      pltpu.make_async_copy(tbl.at[pl.ds(0, 128), :],
                            rows_v.at[buf, pl.ds(k * 128, 128), :],
                            sem).wait()

  def compute_store(o_hbm, c, buf):
    def row(r, carry):
      base = r * S
      acc0 = rows_v[buf, base, pl.ds(0, 16)]
      acc1 = rows_v[buf, base, pl.ds(16, 16)]
      for s in range(1, S):
        acc0 = acc0 + rows_v[buf, base + s, pl.ds(0, 16)]
        acc1 = acc1 + rows_v[buf, base + s, pl.ds(16, 16)]
      out_v[r, pl.ds(0, 16)] = acc0
      out_v[r, pl.ds(16, 16)] = acc1
      return carry
    lax.fori_loop(0, CHUNK, row, 0)
    pltpu.sync_copy(out_v, o_hbm.at[pl.ds(wid * RPW + c * CHUNK, CHUNK), :])

  for tbl, o_hbm in ((t0_hbm, o0_hbm), (t1_hbm, o1_hbm), (t2_hbm, o2_hbm)):
    load_idx(0, 0)
    fire(tbl, 0, gsem0)

    def pair(j, carry, tbl=tbl, o_hbm=o_hbm):
      c0 = 2 * j
      load_idx(c0 + 1, 1)
      fire(tbl, 1, gsem1)
      drain(tbl, 0, gsem0)
      compute_store(o_hbm, c0, 0)
      load_idx(c0 + 2, 0)
      fire(tbl, 0, gsem0)
      drain(tbl, 1, gsem1)
      compute_store(o_hbm, c0 + 1, 1)
      return carry

    lax.fori_loop(0, NCH // 2 - 1, pair, 0)
    c0 = NCH - 2
    load_idx(c0 + 1, 1)
    fire(tbl, 1, gsem1)
    drain(tbl, 0, gsem0)
    compute_store(o_hbm, c0, 0)
    drain(tbl, 1, gsem1)
    compute_store(o_hbm, c0 + 1, 1)


@functools.cache
def _seg_gather():
  mesh = plsc.VectorSubcoreMesh(
      core_axis_name="c", subcore_axis_name="s",
      num_cores=NC, num_subcores=NS)
  return pl.kernel(
      _seg_gather_body,
      mesh=mesh,
      out_type=[jax.ShapeDtypeStruct((ROWS, E), jnp.float32)] * 3,
      scratch_types=[
          pltpu.VMEM((2, IPC), jnp.int32),
          pltpu.VMEM((2, IPC, E), jnp.float32),
          pltpu.VMEM((CHUNK, E), jnp.float32),
          pltpu.SemaphoreType.DMA,
          pltpu.SemaphoreType.DMA,
      ],
      compiler_params=pltpu.CompilerParams(use_tc_tiling_on_sc=False),
  )


def _hops_body(g1, g2, g3, o):
  g1v = g1[...]
  q1 = jnp.sum(g1v, axis=1) * (1.0 / M)
  p1 = jnp.sum(g1v * q1[:, None, :], axis=2)
  a1 = jax.nn.softmax(p1, axis=1)
  g2v = g2[...]
  q2 = q1 + jnp.sum(a1[:, :, None] * g2v, axis=1)
  p2 = jnp.sum(g2v * q2[:, None, :], axis=2)
  a2 = jax.nn.softmax(p2, axis=1)
  o[...] = q2 + jnp.sum(a2[:, :, None] * g3[...], axis=1)


def _hops(G1, G2, G3):
  BB = 256
  spec3 = pl.BlockSpec((BB, M, E), lambda i: (i, 0, 0))
  return pl.pallas_call(
      _hops_body,
      grid=(B // BB,),
      in_specs=[spec3, spec3, spec3],
      out_specs=pl.BlockSpec((BB, E), lambda i: (i, 0)),
      out_shape=jax.ShapeDtypeStruct((B, E), jnp.float32),
  )(G1, G2, G3)


def kernel(context, A0, A1, A2, C_last):
  del A0
  ctx = context.reshape(-1)
  G1, G2, G3 = _seg_gather()(ctx, A1, A2, C_last)
  return _hops(G1.reshape(B, M, E), G2.reshape(B, M, E), G3.reshape(B, M, E))
